# baseline (speedup 1.0000x reference)
import math

import jax
import jax.numpy as jnp
import numpy as np
from jax.experimental import pallas as pl
from jax.experimental.pallas import tpu as pltpu

B = 4
S = 1024
H = 16
KVH = 4
G = H // KVH
D = 128
WINDOW = 512
SCALE = 0.08838834764831845

BQ = 128
KS = BQ + WINDOW
NQ = S // BQ


def _slopes(n):
    def pow2(n):
        start = 2 ** (-(2 ** (-(math.log2(n) - 3))))
        return [start * start ** i for i in range(n)]
    if math.log2(n).is_integer():
        return pow2(n)
    closest = 2 ** math.floor(math.log2(n))
    return pow2(closest) + _slopes(2 * closest)[0::2][: n - closest]


def _attn_kernel(slopes_ref, q_ref, k_ref, v_ref, o_ref):
    h = pl.program_id(1)
    qi = pl.program_id(2)
    q_base = qi * BQ
    start = pl.multiple_of(jnp.maximum(q_base + BQ - KS, 0), BQ)

    kspan = k_ref[0, pl.ds(start, KS), :]
    vspan = v_ref[0, pl.ds(start, KS), :]

    i_idx = q_base + jax.lax.broadcasted_iota(jnp.int32, (BQ, KS), 0)
    j_idx = start + jax.lax.broadcasted_iota(jnp.int32, (BQ, KS), 1)
    valid = (j_idx <= i_idx) & (j_idx >= i_idx - WINDOW)
    delta_masked = jnp.where(
        valid, (j_idx - i_idx).astype(jnp.float32), jnp.float32(-1e30))

    for g in range(G):
        qg = q_ref[0, :, g * D:(g + 1) * D] * jnp.float32(SCALE)
        s = jax.lax.dot_general(
            qg, kspan, (((1,), (1,)), ((), ())),
            preferred_element_type=jnp.float32,
        )
        p = jnp.exp(s + slopes_ref[h, g] * delta_masked)
        l = jnp.sum(p, axis=1, keepdims=True)
        og = jax.lax.dot_general(
            p, vspan, (((1,), (0,)), ((), ())),
            preferred_element_type=jnp.float32,
        )
        o_ref[0, :, g * D:(g + 1) * D] = og / l


def kernel(q, k, v):
    qh = q.reshape(B, S, H * D)
    kh = k.reshape(B, S, KVH * D)
    vh = v.reshape(B, S, KVH * D)
    slopes = jnp.asarray(
        np.array(_slopes(H), dtype=np.float32).reshape(KVH, G))

    out = pl.pallas_call(
        _attn_kernel,
        grid=(B, KVH, NQ),
        in_specs=[
            pl.BlockSpec(memory_space=pltpu.SMEM),
            pl.BlockSpec((1, BQ, G * D), lambda b, h, qi: (b, qi, h)),
            pl.BlockSpec((1, S, D), lambda b, h, qi: (b, 0, h)),
            pl.BlockSpec((1, S, D), lambda b, h, qi: (b, 0, h)),
        ],
        out_specs=pl.BlockSpec((1, BQ, G * D), lambda b, h, qi: (b, qi, h)),
        out_shape=jax.ShapeDtypeStruct((B, S, H * D), jnp.float32),
    )(slopes, qh, kh, vh)
    return out.reshape(B * S, H * D)


# speedup vs baseline: 1.1579x; 1.1579x over previous
import math

import jax
import jax.numpy as jnp
import numpy as np
from jax.experimental import pallas as pl
from jax.experimental.pallas import tpu as pltpu

B = 4
S = 1024
H = 16
KVH = 4
G = H // KVH
D = 128
WINDOW = 512
SCALE = 0.08838834764831845

BQ = 256
KS = BQ + WINDOW
BC = 256
NC = KS // BC
NQ = S // BQ


def _slopes(n):
    def pow2(n):
        start = 2 ** (-(2 ** (-(math.log2(n) - 3))))
        return [start * start ** i for i in range(n)]
    if math.log2(n).is_integer():
        return pow2(n)
    closest = 2 ** math.floor(math.log2(n))
    return pow2(closest) + _slopes(2 * closest)[0::2][: n - closest]


def _attn_kernel(slopes_ref, q_ref, k_ref, v_ref, o_ref):
    h = pl.program_id(1)
    qi = pl.program_id(2)
    q_base = qi * BQ
    start = pl.multiple_of(jnp.maximum(q_base + BQ - KS, 0), BQ)

    i_col = q_base + jax.lax.broadcasted_iota(jnp.int32, (BQ, BC), 0)
    qgs = [q_ref[0, :, g * D:(g + 1) * D] * jnp.float32(SCALE)
           for g in range(G)]
    slope = [slopes_ref[h, g] for g in range(G)]

    def chunk_body(c, carry):
        cs = pl.multiple_of(start + c * BC, BC)
        kc = k_ref[0, pl.ds(cs, BC), :]
        vc = v_ref[0, pl.ds(cs, BC), :]
        j_idx = cs + jax.lax.broadcasted_iota(jnp.int32, (BQ, BC), 1)
        valid = (j_idx <= i_col) & (j_idx >= i_col - WINDOW)
        delta_masked = jnp.where(
            valid, (j_idx - i_col).astype(jnp.float32), jnp.float32(-1e30))
        out = []
        for g in range(G):
            o_acc, l_acc = carry[2 * g], carry[2 * g + 1]
            s = jax.lax.dot_general(
                qgs[g], kc, (((1,), (1,)), ((), ())),
                preferred_element_type=jnp.float32,
            )
            p = jnp.exp(s + slope[g] * delta_masked)
            l_acc = l_acc + jnp.sum(p, axis=1, keepdims=True)
            o_acc = o_acc + jax.lax.dot_general(
                p, vc, (((1,), (0,)), ((), ())),
                preferred_element_type=jnp.float32,
            )
            out.extend([o_acc, l_acc])
        return tuple(out)

    n_chunks = jnp.minimum(qi + 1, NC)
    init = []
    for g in range(G):
        init.extend([jnp.zeros((BQ, D), jnp.float32),
                     jnp.zeros((BQ, 1), jnp.float32)])
    carry = jax.lax.fori_loop(0, n_chunks, chunk_body, tuple(init))
    for g in range(G):
        o_ref[0, :, g * D:(g + 1) * D] = carry[2 * g] / carry[2 * g + 1]


def kernel(q, k, v):
    qh = q.reshape(B, S, H * D)
    kh = k.reshape(B, S, KVH * D)
    vh = v.reshape(B, S, KVH * D)
    slopes = jnp.asarray(
        np.array(_slopes(H), dtype=np.float32).reshape(KVH, G))

    out = pl.pallas_call(
        _attn_kernel,
        grid=(B, KVH, NQ),
        in_specs=[
            pl.BlockSpec(memory_space=pltpu.SMEM),
            pl.BlockSpec((1, BQ, G * D), lambda b, h, qi: (b, qi, h)),
            pl.BlockSpec((1, S, D), lambda b, h, qi: (b, 0, h)),
            pl.BlockSpec((1, S, D), lambda b, h, qi: (b, 0, h)),
        ],
        out_specs=pl.BlockSpec((1, BQ, G * D), lambda b, h, qi: (b, qi, h)),
        out_shape=jax.ShapeDtypeStruct((B, S, H * D), jnp.float32),
    )(slopes, qh, kh, vh)
    return out.reshape(B * S, H * D)


# speedup vs baseline: 1.6482x; 1.4234x over previous
import functools
import math

import jax
import jax.numpy as jnp
import numpy as np
from jax.experimental import pallas as pl
from jax.experimental.pallas import tpu as pltpu

B = 4
S = 1024
H = 16
KVH = 4
G = H // KVH
D = 128
WINDOW = 512
SCALE = 0.08838834764831845

BQ = 256


def _slopes(n):
    def pow2(n):
        start = 2 ** (-(2 ** (-(math.log2(n) - 3))))
        return [start * start ** i for i in range(n)]
    if math.log2(n).is_integer():
        return pow2(n)
    closest = 2 ** math.floor(math.log2(n))
    return pow2(closest) + _slopes(2 * closest)[0::2][: n - closest]


def _attn_body(q_ref, k_ref, v_ref, o_ref, slopes_ref, *, ks, qi_off):
    h = pl.program_id(1)
    qi = pl.program_id(2) + qi_off
    q_base = qi * BQ
    start = pl.multiple_of(jnp.maximum(q_base + BQ - ks, 0), BQ)

    kspan = k_ref[0, pl.ds(start, ks), :]
    vspan = v_ref[0, pl.ds(start, ks), :]
    i_idx = q_base + jax.lax.broadcasted_iota(jnp.int32, (BQ, ks), 0)
    j_idx = start + jax.lax.broadcasted_iota(jnp.int32, (BQ, ks), 1)
    valid = (j_idx <= i_idx) & (j_idx >= i_idx - WINDOW)
    delta_masked = jnp.where(
        valid, (j_idx - i_idx).astype(jnp.float32), jnp.float32(-1e30))

    for g in range(G):
        qg = q_ref[0, :, g * D:(g + 1) * D] * jnp.float32(SCALE)
        s = jax.lax.dot_general(
            qg, kspan, (((1,), (1,)), ((), ())),
            preferred_element_type=jnp.float32,
        )
        p = jnp.exp(s + slopes_ref[h, g] * delta_masked)
        l = jnp.sum(p, axis=1, keepdims=True)
        og = jax.lax.dot_general(
            p, vspan, (((1,), (0,)), ((), ())),
            preferred_element_type=jnp.float32,
        )
        o_ref[0, :, g * D:(g + 1) * D] = og * (1.0 / l)


def _low_kernel(slopes_ref, q_ref, k_ref, v_ref, o_ref):
    _attn_body(q_ref, k_ref, v_ref, o_ref, slopes_ref, ks=512, qi_off=0)


def _high_kernel(slopes_ref, q_ref, k_ref, v_ref, _prev_ref, o_ref):
    _attn_body(q_ref, k_ref, v_ref, o_ref, slopes_ref, ks=768, qi_off=2)


def kernel(q, k, v):
    qh = q.reshape(B, S, H * D)
    kh = k.reshape(B, S, KVH * D)
    vh = v.reshape(B, S, KVH * D)
    slopes = jnp.asarray(
        np.array(_slopes(H), dtype=np.float32).reshape(KVH, G))

    out_shape = jax.ShapeDtypeStruct((B, S, H * D), jnp.float32)
    qo_spec = pl.BlockSpec((1, BQ, G * D), lambda b, h, qi: (b, qi, h))
    qo_spec_hi = pl.BlockSpec((1, BQ, G * D), lambda b, h, qi: (b, qi + 2, h))
    kv_spec = pl.BlockSpec((1, S, D), lambda b, h, qi: (b, 0, h))

    o1 = pl.pallas_call(
        _low_kernel,
        grid=(B, KVH, 2),
        in_specs=[
            pl.BlockSpec(memory_space=pltpu.SMEM),
            qo_spec, kv_spec, kv_spec,
        ],
        out_specs=qo_spec,
        out_shape=out_shape,
    )(slopes, qh, kh, vh)

    out = pl.pallas_call(
        _high_kernel,
        grid=(B, KVH, 2),
        in_specs=[
            pl.BlockSpec(memory_space=pltpu.SMEM),
            qo_spec_hi, kv_spec, kv_spec,
            pl.BlockSpec(memory_space=pltpu.MemorySpace.HBM),
        ],
        out_specs=qo_spec_hi,
        out_shape=out_shape,
        input_output_aliases={4: 0},
    )(slopes, qh, kh, vh, o1)
    return out.reshape(B * S, H * D)


# speedup vs baseline: 1.7219x; 1.0447x over previous
import math

import jax
import jax.numpy as jnp
import numpy as np
from jax.experimental import pallas as pl
from jax.experimental.pallas import tpu as pltpu

B = 4
S = 1024
H = 16
KVH = 4
G = H // KVH
D = 128
WINDOW = 512
SCALE = 0.08838834764831845

BQ = 256
KS = BQ + WINDOW
BC = 256
NC = KS // BC
NQ = S // BQ


def _slopes(n):
    def pow2(n):
        start = 2 ** (-(2 ** (-(math.log2(n) - 3))))
        return [start * start ** i for i in range(n)]
    if math.log2(n).is_integer():
        return pow2(n)
    closest = 2 ** math.floor(math.log2(n))
    return pow2(closest) + _slopes(2 * closest)[0::2][: n - closest]


def _attn_kernel(slopes_ref, q_ref, k_ref, v_ref, o_ref):
    h = pl.program_id(1)
    qi = pl.program_id(2)
    q_base = qi * BQ
    start = pl.multiple_of(jnp.maximum(q_base + BQ - KS, 0), BQ)

    kspan = k_ref[0, pl.ds(start, KS), :]
    vspan = v_ref[0, pl.ds(start, KS), :]
    i_idx = q_base + jax.lax.broadcasted_iota(jnp.int32, (BQ, KS), 0)
    j_idx = start + jax.lax.broadcasted_iota(jnp.int32, (BQ, KS), 1)
    valid = (j_idx <= i_idx) & (j_idx >= i_idx - WINDOW)
    delta_masked = jnp.where(
        valid, (j_idx - i_idx).astype(jnp.float32), jnp.float32(-1e30))

    for g in range(G):
        qg = q_ref[0, :, g * D:(g + 1) * D] * jnp.float32(SCALE)
        s = jax.lax.dot_general(
            qg, kspan, (((1,), (1,)), ((), ())),
            preferred_element_type=jnp.float32,
        )
        p = jnp.exp(s + slopes_ref[h, g] * delta_masked)
        l = jnp.sum(p, axis=1, keepdims=True)
        og = jax.lax.dot_general(
            p, vspan, (((1,), (0,)), ((), ())),
            preferred_element_type=jnp.float32,
        )
        o_ref[0, :, g * D:(g + 1) * D] = og * (1.0 / l)


def kernel(q, k, v):
    qh = q.reshape(B, S, H * D)
    kh = k.reshape(B, S, KVH * D)
    vh = v.reshape(B, S, KVH * D)
    slopes = jnp.asarray(
        np.array(_slopes(H), dtype=np.float32).reshape(KVH, G))

    out = pl.pallas_call(
        _attn_kernel,
        grid=(B, KVH, NQ),
        in_specs=[
            pl.BlockSpec(memory_space=pltpu.SMEM),
            pl.BlockSpec((1, BQ, G * D), lambda b, h, qi: (b, qi, h)),
            pl.BlockSpec((1, S, D), lambda b, h, qi: (b, 0, h)),
            pl.BlockSpec((1, S, D), lambda b, h, qi: (b, 0, h)),
        ],
        out_specs=pl.BlockSpec((1, BQ, G * D), lambda b, h, qi: (b, qi, h)),
        out_shape=jax.ShapeDtypeStruct((B, S, H * D), jnp.float32),
    )(slopes, qh, kh, vh)
    return out.reshape(B * S, H * D)
